# baseline (speedup 1.0000x reference)
import functools

import jax
import jax.numpy as jnp
from jax import lax
from jax.experimental import pallas as pl
from jax.experimental.pallas import tpu as pltpu
from jax.experimental.pallas import tpu_sc as plsc

_NC = 2
_NS = 16
_NW = _NC * _NS
_K = 128
_GAR = 240


def _sc_mesh():
    return plsc.VectorSubcoreMesh(
        core_axis_name="c", subcore_axis_name="s",
        num_cores=_NC, num_subcores=_NS)


def _degree_sc(ei3, n):
    r_tot = ei3.shape[0]
    base = r_tot // _NW
    rem = r_tot % _NW
    hrows = (base + 2) // 2
    n_acc = n + _GAR
    half = n_acc // 2
    hbuf = (half + 15) // 16 * 16
    obase = hbuf
    obuf = (_K + 15) // 16 * 16

    @functools.partial(
        pl.kernel,
        out_type=(jax.ShapeDtypeStruct((n_acc,), jnp.float32),
                  jax.ShapeDtypeStruct((n_acc,), jnp.float32)),
        mesh=_sc_mesh(),
        scratch_types=[
            pltpu.VMEM((hrows, 8, _K), jnp.int32),
            pltpu.VMEM((hbuf + obuf,), jnp.float32),
            pltpu.VMEM_SHARED((n_acc,), jnp.float32),
        ],
    )
    def run(ei_hbm, deg0_hbm, deg1_hbm, idx_v, ones_v, deg_sh):
        c = lax.axis_index("c")
        s = lax.axis_index("s")
        w = s * _NC + c
        rw_w = base + jnp.where(w < rem, 1, 0)
        start = base * w + jnp.minimum(w, rem)

        def fill(i, carry):
            ones_v[pl.ds(pl.multiple_of(i * 16, 16), 16)] = jnp.full(
                (16,), 0.5, jnp.float32)
            return carry
        lax.fori_loop(0, hbuf // 16, fill, 0)

        def fill1(i, carry):
            ones_v[pl.ds(pl.multiple_of(obase + i * 16, 16), 16)] = jnp.ones(
                (16,), jnp.float32)
            return carry
        lax.fori_loop(0, obuf // 16, fill1, 0)

        @pl.when(s == 0)
        def _():
            pltpu.sync_copy(ones_v.at[pl.ds(0, half)], deg_sh.at[pl.ds(0, half)])

        @pl.when(s == 1)
        def _():
            pltpu.sync_copy(ones_v.at[pl.ds(0, half)],
                            deg_sh.at[pl.ds(half, half)])

        plsc.subcore_barrier()

        ones_k = ones_v.at[pl.ds(obase, _K)]

        for h in range(2):
            off = start if h == 0 else start + rw_w - hrows
            r0 = 0 if h == 0 else 2 * hrows - rw_w
            pltpu.sync_copy(ei_hbm.at[pl.ds(off, hrows)], idx_v)

            def step(r, carry):
                for q in range(4):
                    pltpu.sync_copy(ones_k, deg_sh.at[idx_v.at[r, 2 * q + 1]],
                                    add=True)
                return carry
            lax.fori_loop(r0, hrows, step, 0)

        plsc.subcore_barrier()

        @pl.when(jnp.logical_and(s == 0, c == 0))
        def _():
            pltpu.sync_copy(deg_sh, deg0_hbm)

        @pl.when(jnp.logical_and(s == 0, c == 1))
        def _():
            pltpu.sync_copy(deg_sh, deg1_hbm)

    return run(ei3)


def _segsum_sc(ei3, y):
    r_tot = ei3.shape[0]
    base = r_tot // _NW
    rem = r_tot % _NW
    hrows = (base + 2) // 2
    n, d = y.shape
    rpt = -(-n // (_NS * 128)) * 128
    n_pad = rpt * _NS

    @functools.partial(
        pl.kernel,
        out_type=jax.ShapeDtypeStruct((_NC, n_pad, d), jnp.float32),
        mesh=_sc_mesh(),
        scratch_types=[
            pltpu.VMEM((hrows, 8, _K), jnp.int32),
            pltpu.VMEM((2, _K, d), jnp.float32),
            pltpu.VMEM_SHARED((n_pad, d), jnp.float32),
            pltpu.SemaphoreType.DMA,
            pltpu.SemaphoreType.DMA,
        ],
    )
    def run(ei_hbm, y_hbm, out_hbm, idx_v, rows_v, acc_sh, sem0, sem1):
        c = lax.axis_index("c")
        s = lax.axis_index("s")
        w = s * _NC + c
        rw_w = base + jnp.where(w < rem, 1, 0)
        start = base * w + jnp.minimum(w, rem)

        def zrow(i, carry):
            for jj in range(8):
                rows_v[0, i, pl.ds(jj * 16, 16)] = jnp.zeros((16,), jnp.float32)
            return carry
        lax.fori_loop(0, _K, zrow, 0)
        for k in range(rpt // 128):
            pltpu.sync_copy(rows_v.at[0],
                            acc_sh.at[pl.ds(s * rpt + k * 128, 128)])
        plsc.subcore_barrier()

        gb = (rows_v.at[0], rows_v.at[1])
        sems = (sem0, sem1)
        for h in range(2):
            off = start if h == 0 else start + rw_w - hrows
            r0 = 0 if h == 0 else 2 * hrows - rw_w
            pltpu.sync_copy(ei_hbm.at[pl.ds(off, hrows)], idx_v)

            pltpu.async_copy(y_hbm.at[idx_v.at[r0, 0]], gb[0], sems[0])
            pltpu.async_copy(y_hbm.at[idx_v.at[r0, 2]], gb[1], sems[1])

            def step(r, carry):
                for q in range(4):
                    b = q % 2
                    pltpu.make_async_copy(
                        y_hbm.at[idx_v.at[r, 2 * q]], gb[b], sems[b]).wait()
                    pltpu.sync_copy(gb[b], acc_sh.at[idx_v.at[r, 2 * q + 1]],
                                    add=True)
                    if q < 2:
                        pltpu.async_copy(
                            y_hbm.at[idx_v.at[r, 2 * q + 4]], gb[b], sems[b])
                    else:

                        @pl.when(r + 1 < hrows)
                        def _():
                            pltpu.async_copy(
                                y_hbm.at[idx_v.at[r + 1, 2 * q - 4]],
                                gb[b], sems[b])
                return carry
            lax.fori_loop(r0, hrows, step, 0)

        plsc.subcore_barrier()
        pltpu.sync_copy(acc_sh.at[pl.ds(s * rpt, rpt)],
                        out_hbm.at[c, pl.ds(s * rpt, rpt)])

    return run(ei3, y)


_BR = 1000


def _mm_scale_tc(x, w, deg_t):
    n, d = x.shape

    def body(x_ref, w_ref, deg_ref, o_ref):
        dinv = lax.rsqrt(deg_ref[:, :1] + deg_ref[:, 1:2])
        y = jnp.dot(x_ref[...], w_ref[...], preferred_element_type=jnp.float32)
        o_ref[...] = y * dinv

    return pl.pallas_call(
        body,
        grid=(n // _BR,),
        in_specs=[
            pl.BlockSpec((_BR, d), lambda i: (i, 0)),
            pl.BlockSpec((d, d), lambda i: (0, 0)),
            pl.BlockSpec((_BR, 2), lambda i: (i, 0)),
        ],
        out_specs=pl.BlockSpec((_BR, d), lambda i: (i, 0)),
        out_shape=jax.ShapeDtypeStruct((n, d), jnp.float32),
    )(x, w, deg_t)


def _mid_layer_tc(acc, y1, deg_t, b, w):
    n, d = y1.shape

    def body(acc_ref, y1_ref, deg_ref, b_ref, w_ref, o_ref):
        dinv = lax.rsqrt(deg_ref[:, :1] + deg_ref[:, 1:2])
        tot = acc_ref[0] + acc_ref[1] + y1_ref[...]
        h = jnp.maximum(tot * dinv + b_ref[...], 0.0)
        y2 = jnp.dot(h, w_ref[...], preferred_element_type=jnp.float32)
        o_ref[...] = y2 * dinv

    return pl.pallas_call(
        body,
        grid=(n // _BR,),
        in_specs=[
            pl.BlockSpec((_NC, _BR, d), lambda i: (0, i, 0)),
            pl.BlockSpec((_BR, d), lambda i: (i, 0)),
            pl.BlockSpec((_BR, 2), lambda i: (i, 0)),
            pl.BlockSpec((1, d), lambda i: (0, 0)),
            pl.BlockSpec((d, d), lambda i: (0, 0)),
        ],
        out_specs=pl.BlockSpec((_BR, d), lambda i: (i, 0)),
        out_shape=jax.ShapeDtypeStruct((n, d), jnp.float32),
    )(acc, y1, deg_t, b, w)


def _final_tc(acc, y2, deg_t, b):
    n, d = y2.shape

    def body(acc_ref, y2_ref, deg_ref, b_ref, o_ref):
        dinv = lax.rsqrt(deg_ref[:, :1] + deg_ref[:, 1:2])
        tot = acc_ref[0] + acc_ref[1] + y2_ref[...]
        o_ref[...] = jnp.maximum(tot * dinv + b_ref[...], 0.0)

    return pl.pallas_call(
        body,
        grid=(n // _BR,),
        in_specs=[
            pl.BlockSpec((_NC, _BR, d), lambda i: (0, i, 0)),
            pl.BlockSpec((_BR, d), lambda i: (i, 0)),
            pl.BlockSpec((_BR, 2), lambda i: (i, 0)),
            pl.BlockSpec((1, d), lambda i: (0, 0)),
        ],
        out_specs=pl.BlockSpec((_BR, d), lambda i: (i, 0)),
        out_shape=jax.ShapeDtypeStruct((n, d), jnp.float32),
    )(acc, y2, deg_t, b)


def kernel(edge_index, x, W1, b1, W2, b2):
    n, d = x.shape
    e = edge_index.shape[1]

    assert e % (4 * _K) == 0
    assert n % _NS == 0 and ((n + _GAR) // 2) % 8 == 0 and n % _BR == 0
    ei3 = (edge_index.reshape(2, e // _K, _K)
           .transpose(1, 0, 2)
           .reshape(e // (4 * _K), 8, _K))

    deg0, deg1 = _degree_sc(ei3, n)
    deg_t = jnp.stack([deg0[:n], deg1[:n]], axis=1)
    b1r = b1.reshape(1, d)
    b2r = b2.reshape(1, d)

    y1 = _mm_scale_tc(x, W1, deg_t)
    acc1 = _segsum_sc(ei3, y1)
    y2 = _mid_layer_tc(acc1, y1, deg_t, b1r, W2)
    acc2 = _segsum_sc(ei3, y2)
    return _final_tc(acc2, y2, deg_t, b2r)


# speedup vs baseline: 1.0320x; 1.0320x over previous
import functools

import jax
import jax.numpy as jnp
from jax import lax
from jax.experimental import pallas as pl
from jax.experimental.pallas import tpu as pltpu
from jax.experimental.pallas import tpu_sc as plsc

_NC = 2
_NS = 16
_NW = _NC * _NS
_K = 128
_GAR = 240


def _sc_mesh():
    return plsc.VectorSubcoreMesh(
        core_axis_name="c", subcore_axis_name="s",
        num_cores=_NC, num_subcores=_NS)


def _degree_sc(ei3, n):
    rw = ei3.shape[0] // _NW
    n_acc = n + _GAR
    half = n_acc // 2
    hbuf = (half + 15) // 16 * 16
    obase = hbuf
    obuf = (_K + 15) // 16 * 16

    @functools.partial(
        pl.kernel,
        out_type=(jax.ShapeDtypeStruct((n_acc,), jnp.float32),
                  jax.ShapeDtypeStruct((n_acc,), jnp.float32)),
        mesh=_sc_mesh(),
        scratch_types=[
            pltpu.VMEM((rw, 8, _K), jnp.int32),
            pltpu.VMEM((hbuf + obuf,), jnp.float32),
            pltpu.VMEM_SHARED((n_acc,), jnp.float32),
        ],
    )
    def run(ei_hbm, deg0_hbm, deg1_hbm, idx_v, ones_v, deg_sh):
        c = lax.axis_index("c")
        s = lax.axis_index("s")
        w = s * _NC + c

        def fill(i, carry):
            ones_v[pl.ds(pl.multiple_of(i * 16, 16), 16)] = jnp.full(
                (16,), 0.5, jnp.float32)
            return carry
        lax.fori_loop(0, hbuf // 16, fill, 0)

        def fill1(i, carry):
            ones_v[pl.ds(pl.multiple_of(obase + i * 16, 16), 16)] = jnp.ones(
                (16,), jnp.float32)
            return carry
        lax.fori_loop(0, obuf // 16, fill1, 0)

        @pl.when(s == 0)
        def _():
            pltpu.sync_copy(ones_v.at[pl.ds(0, half)], deg_sh.at[pl.ds(0, half)])

        @pl.when(s == 1)
        def _():
            pltpu.sync_copy(ones_v.at[pl.ds(0, half)],
                            deg_sh.at[pl.ds(half, half)])

        pltpu.sync_copy(ei_hbm.at[pl.ds(w * rw, rw)], idx_v)
        plsc.subcore_barrier()

        ones_k = ones_v.at[pl.ds(obase, _K)]

        def step(r, carry):
            for q in range(4):
                pltpu.sync_copy(ones_k, deg_sh.at[idx_v.at[r, 2 * q + 1]],
                                add=True)
            return carry
        lax.fori_loop(0, rw, step, 0)

        plsc.subcore_barrier()

        @pl.when(jnp.logical_and(s == 0, c == 0))
        def _():
            pltpu.sync_copy(deg_sh, deg0_hbm)

        @pl.when(jnp.logical_and(s == 0, c == 1))
        def _():
            pltpu.sync_copy(deg_sh, deg1_hbm)

    return run(ei3)


def _segsum_sc(ei3, y):
    rw = ei3.shape[0] // _NW
    n, d = y.shape
    rpt = -(-n // (_NS * 128)) * 128
    n_pad = rpt * _NS

    hrows = rw // 2

    @functools.partial(
        pl.kernel,
        out_type=jax.ShapeDtypeStruct((_NC, n_pad, d), jnp.float32),
        mesh=_sc_mesh(),
        scratch_types=[
            pltpu.VMEM((hrows, 8, _K), jnp.int32),
            pltpu.VMEM((2, _K, d), jnp.float32),
            pltpu.VMEM_SHARED((n_pad, d), jnp.float32),
            pltpu.SemaphoreType.DMA,
            pltpu.SemaphoreType.DMA,
        ],
    )
    def run(ei_hbm, y_hbm, out_hbm, idx_v, rows_v, acc_sh, sem0, sem1):
        c = lax.axis_index("c")
        s = lax.axis_index("s")
        w = s * _NC + c

        def zrow(i, carry):
            for jj in range(8):
                rows_v[0, i, pl.ds(jj * 16, 16)] = jnp.zeros((16,), jnp.float32)
            return carry
        lax.fori_loop(0, _K, zrow, 0)
        for k in range(rpt // 128):
            pltpu.sync_copy(rows_v.at[0],
                            acc_sh.at[pl.ds(s * rpt + k * 128, 128)])
        plsc.subcore_barrier()

        gb = (rows_v.at[0], rows_v.at[1])
        sems = (sem0, sem1)
        for h in range(2):
            pltpu.sync_copy(ei_hbm.at[pl.ds(w * rw + h * hrows, hrows)], idx_v)

            pltpu.async_copy(y_hbm.at[idx_v.at[0, 0]], gb[0], sems[0])
            pltpu.async_copy(y_hbm.at[idx_v.at[0, 2]], gb[1], sems[1])

            def step(r, carry):
                for q in range(4):
                    b = q % 2
                    pltpu.make_async_copy(
                        y_hbm.at[idx_v.at[r, 2 * q]], gb[b], sems[b]).wait()
                    pltpu.sync_copy(gb[b], acc_sh.at[idx_v.at[r, 2 * q + 1]],
                                    add=True)
                    if q < 2:
                        pltpu.async_copy(
                            y_hbm.at[idx_v.at[r, 2 * q + 4]], gb[b], sems[b])
                    else:

                        @pl.when(r + 1 < hrows)
                        def _():
                            pltpu.async_copy(
                                y_hbm.at[idx_v.at[r + 1, 2 * q - 4]],
                                gb[b], sems[b])
                return carry
            lax.fori_loop(0, hrows, step, 0)

        plsc.subcore_barrier()
        pltpu.sync_copy(acc_sh.at[pl.ds(s * rpt, rpt)],
                        out_hbm.at[c, pl.ds(s * rpt, rpt)])

    return run(ei3, y)


_BR = 2000


def _mm_scale_tc(x, w, deg_t):
    n, d = x.shape

    def body(x_ref, w_ref, deg_ref, o_ref):
        dinv = lax.rsqrt(deg_ref[:, :1] + deg_ref[:, 1:2])
        y = jnp.dot(x_ref[...], w_ref[...], preferred_element_type=jnp.float32)
        o_ref[...] = y * dinv

    return pl.pallas_call(
        body,
        grid=(n // _BR,),
        in_specs=[
            pl.BlockSpec((_BR, d), lambda i: (i, 0)),
            pl.BlockSpec((d, d), lambda i: (0, 0)),
            pl.BlockSpec((_BR, 2), lambda i: (i, 0)),
        ],
        out_specs=pl.BlockSpec((_BR, d), lambda i: (i, 0)),
        out_shape=jax.ShapeDtypeStruct((n, d), jnp.float32),
    )(x, w, deg_t)


def _mid_layer_tc(acc, y1, deg_t, b, w):
    n, d = y1.shape

    def body(acc_ref, y1_ref, deg_ref, b_ref, w_ref, o_ref):
        dinv = lax.rsqrt(deg_ref[:, :1] + deg_ref[:, 1:2])
        tot = acc_ref[0] + acc_ref[1] + y1_ref[...]
        h = jnp.maximum(tot * dinv + b_ref[...], 0.0)
        y2 = jnp.dot(h, w_ref[...], preferred_element_type=jnp.float32)
        o_ref[...] = y2 * dinv

    return pl.pallas_call(
        body,
        grid=(n // _BR,),
        in_specs=[
            pl.BlockSpec((_NC, _BR, d), lambda i: (0, i, 0)),
            pl.BlockSpec((_BR, d), lambda i: (i, 0)),
            pl.BlockSpec((_BR, 2), lambda i: (i, 0)),
            pl.BlockSpec((1, d), lambda i: (0, 0)),
            pl.BlockSpec((d, d), lambda i: (0, 0)),
        ],
        out_specs=pl.BlockSpec((_BR, d), lambda i: (i, 0)),
        out_shape=jax.ShapeDtypeStruct((n, d), jnp.float32),
    )(acc, y1, deg_t, b, w)


def _final_tc(acc, y2, deg_t, b):
    n, d = y2.shape

    def body(acc_ref, y2_ref, deg_ref, b_ref, o_ref):
        dinv = lax.rsqrt(deg_ref[:, :1] + deg_ref[:, 1:2])
        tot = acc_ref[0] + acc_ref[1] + y2_ref[...]
        o_ref[...] = jnp.maximum(tot * dinv + b_ref[...], 0.0)

    return pl.pallas_call(
        body,
        grid=(n // _BR,),
        in_specs=[
            pl.BlockSpec((_NC, _BR, d), lambda i: (0, i, 0)),
            pl.BlockSpec((_BR, d), lambda i: (i, 0)),
            pl.BlockSpec((_BR, 2), lambda i: (i, 0)),
            pl.BlockSpec((1, d), lambda i: (0, 0)),
        ],
        out_specs=pl.BlockSpec((_BR, d), lambda i: (i, 0)),
        out_shape=jax.ShapeDtypeStruct((n, d), jnp.float32),
    )(acc, y2, deg_t, b)


def kernel(edge_index, x, W1, b1, W2, b2):
    n, d = x.shape
    e = edge_index.shape[1]

    nch = 4 * -(-e // (_NW * _K * 4))
    e_pad = _NW * nch * _K
    assert n % _NS == 0 and ((n + _GAR) // 2) % 8 == 0 and n % _BR == 0

    ar = jnp.arange(e_pad - e, dtype=edge_index.dtype)
    pad_blk = jnp.stack([ar % n, n + ar % _GAR], axis=0)
    ei_p = jnp.concatenate([edge_index, pad_blk], axis=1)
    ei3 = (ei_p.reshape(2, e_pad // _K, _K)
           .transpose(1, 0, 2)
           .reshape(e_pad // (4 * _K), 8, _K))

    deg0, deg1 = _degree_sc(ei3, n)
    deg_t = jnp.stack([deg0[:n], deg1[:n]], axis=1)
    b1r = b1.reshape(1, d)
    b2r = b2.reshape(1, d)

    y1 = _mm_scale_tc(x, W1, deg_t)
    acc1 = _segsum_sc(ei3, y1)
    y2 = _mid_layer_tc(acc1, y1, deg_t, b1r, W2)
    acc2 = _segsum_sc(ei3, y2)
    return _final_tc(acc2, y2, deg_t, b2r)


# speedup vs baseline: 1.0346x; 1.0026x over previous
import functools

import jax
import jax.numpy as jnp
from jax import lax
from jax.experimental import pallas as pl
from jax.experimental.pallas import tpu as pltpu
from jax.experimental.pallas import tpu_sc as plsc

_NC = 2
_NS = 16
_NW = _NC * _NS
_K = 128
_GAR = 240


def _sc_mesh():
    return plsc.VectorSubcoreMesh(
        core_axis_name="c", subcore_axis_name="s",
        num_cores=_NC, num_subcores=_NS)


def _degree_sc(ei3, n):
    rw = ei3.shape[0] // _NW
    n_acc = n + _GAR
    half = n_acc // 2
    hbuf = (half + 15) // 16 * 16
    obase = hbuf
    obuf = (_K + 15) // 16 * 16

    @functools.partial(
        pl.kernel,
        out_type=(jax.ShapeDtypeStruct((n_acc,), jnp.float32),
                  jax.ShapeDtypeStruct((n_acc,), jnp.float32)),
        mesh=_sc_mesh(),
        scratch_types=[
            pltpu.VMEM((rw, 8, _K), jnp.int32),
            pltpu.VMEM((hbuf + obuf,), jnp.float32),
            pltpu.VMEM_SHARED((n_acc,), jnp.float32),
        ],
    )
    def run(ei_hbm, deg0_hbm, deg1_hbm, idx_v, ones_v, deg_sh):
        c = lax.axis_index("c")
        s = lax.axis_index("s")
        w = s * _NC + c

        def fill(i, carry):
            ones_v[pl.ds(pl.multiple_of(i * 16, 16), 16)] = jnp.full(
                (16,), 0.5, jnp.float32)
            return carry
        lax.fori_loop(0, hbuf // 16, fill, 0)

        def fill1(i, carry):
            ones_v[pl.ds(pl.multiple_of(obase + i * 16, 16), 16)] = jnp.ones(
                (16,), jnp.float32)
            return carry
        lax.fori_loop(0, obuf // 16, fill1, 0)

        @pl.when(s == 0)
        def _():
            pltpu.sync_copy(ones_v.at[pl.ds(0, half)], deg_sh.at[pl.ds(0, half)])

        @pl.when(s == 1)
        def _():
            pltpu.sync_copy(ones_v.at[pl.ds(0, half)],
                            deg_sh.at[pl.ds(half, half)])

        pltpu.sync_copy(ei_hbm.at[pl.ds(w * rw, rw)], idx_v)
        plsc.subcore_barrier()

        ones_k = ones_v.at[pl.ds(obase, _K)]

        def step(r, carry):
            for q in range(4):
                pltpu.sync_copy(ones_k, deg_sh.at[idx_v.at[r, 2 * q + 1]],
                                add=True)
            return carry
        lax.fori_loop(0, rw, step, 0)

        plsc.subcore_barrier()

        @pl.when(jnp.logical_and(s == 0, c == 0))
        def _():
            pltpu.sync_copy(deg_sh, deg0_hbm)

        @pl.when(jnp.logical_and(s == 0, c == 1))
        def _():
            pltpu.sync_copy(deg_sh, deg1_hbm)

    return run(ei3)


def _segsum_sc(ei3, y):
    rw = ei3.shape[0] // _NW
    n, d = y.shape
    rpt = -(-n // (_NS * 128)) * 128
    n_pad = rpt * _NS

    hrows = rw // 2

    @functools.partial(
        pl.kernel,
        out_type=jax.ShapeDtypeStruct((_NC, n_pad, d), jnp.float32),
        mesh=_sc_mesh(),
        scratch_types=[
            pltpu.VMEM((hrows, 8, _K), jnp.int32),
            pltpu.VMEM((2, _K, d), jnp.float32),
            pltpu.VMEM_SHARED((n_pad, d), jnp.float32),
            pltpu.SemaphoreType.DMA,
            pltpu.SemaphoreType.DMA,
        ],
    )
    def run(ei_hbm, y_hbm, out_hbm, idx_v, rows_v, acc_sh, sem0, sem1):
        c = lax.axis_index("c")
        s = lax.axis_index("s")
        w = s * _NC + c

        def zrow(i, carry):
            for jj in range(8):
                rows_v[0, i, pl.ds(jj * 16, 16)] = jnp.zeros((16,), jnp.float32)
            return carry
        lax.fori_loop(0, _K, zrow, 0)
        for k in range(rpt // 128):
            pltpu.sync_copy(rows_v.at[0],
                            acc_sh.at[pl.ds(s * rpt + k * 128, 128)])
        plsc.subcore_barrier()

        gb = (rows_v.at[0], rows_v.at[1])
        sems = (sem0, sem1)
        for h in range(2):
            pltpu.sync_copy(ei_hbm.at[pl.ds(w * rw + h * hrows, hrows)], idx_v)

            pltpu.async_copy(y_hbm.at[idx_v.at[0, 0]], gb[0], sems[0])
            pltpu.async_copy(y_hbm.at[idx_v.at[0, 2]], gb[1], sems[1])

            def step(r, carry):
                for q in range(4):
                    b = q % 2
                    pltpu.make_async_copy(
                        y_hbm.at[idx_v.at[r, 2 * q]], gb[b], sems[b]).wait()
                    pltpu.sync_copy(gb[b], acc_sh.at[idx_v.at[r, 2 * q + 1]],
                                    add=True)
                    if q < 2:
                        pltpu.async_copy(
                            y_hbm.at[idx_v.at[r, 2 * q + 4]], gb[b], sems[b])
                    else:

                        @pl.when(r + 1 < hrows)
                        def _():
                            pltpu.async_copy(
                                y_hbm.at[idx_v.at[r + 1, 2 * q - 4]],
                                gb[b], sems[b])
                return carry
            lax.fori_loop(0, hrows, step, 0)

        plsc.subcore_barrier()
        pltpu.sync_copy(acc_sh.at[pl.ds(s * rpt, rpt)],
                        out_hbm.at[c, pl.ds(s * rpt, rpt)])

    return run(ei3, y)


_BR = 2000


def _mm_tc(x, w):
    n, d = x.shape

    def body(x_ref, w_ref, o_ref):
        o_ref[...] = jnp.dot(x_ref[...], w_ref[...],
                             preferred_element_type=jnp.float32)

    return pl.pallas_call(
        body,
        grid=(n // _BR,),
        in_specs=[
            pl.BlockSpec((_BR, d), lambda i: (i, 0)),
            pl.BlockSpec((d, d), lambda i: (0, 0)),
        ],
        out_specs=pl.BlockSpec((_BR, d), lambda i: (i, 0)),
        out_shape=jax.ShapeDtypeStruct((n, d), jnp.float32),
    )(x, w)


def _scale_tc(xw, deg_t):
    n, d = xw.shape

    def body(x_ref, deg_ref, o_ref):
        dinv = lax.rsqrt(deg_ref[:, :1] + deg_ref[:, 1:2])
        o_ref[...] = x_ref[...] * dinv

    return pl.pallas_call(
        body,
        grid=(n // _BR,),
        in_specs=[
            pl.BlockSpec((_BR, d), lambda i: (i, 0)),
            pl.BlockSpec((_BR, 2), lambda i: (i, 0)),
        ],
        out_specs=pl.BlockSpec((_BR, d), lambda i: (i, 0)),
        out_shape=jax.ShapeDtypeStruct((n, d), jnp.float32),
    )(xw, deg_t)


def _mid_layer_tc(acc, y1, deg_t, b, w):
    n, d = y1.shape

    def body(acc_ref, y1_ref, deg_ref, b_ref, w_ref, o_ref):
        dinv = lax.rsqrt(deg_ref[:, :1] + deg_ref[:, 1:2])
        tot = acc_ref[0] + acc_ref[1] + y1_ref[...]
        h = jnp.maximum(tot * dinv + b_ref[...], 0.0)
        y2 = jnp.dot(h, w_ref[...], preferred_element_type=jnp.float32)
        o_ref[...] = y2 * dinv

    return pl.pallas_call(
        body,
        grid=(n // _BR,),
        in_specs=[
            pl.BlockSpec((_NC, _BR, d), lambda i: (0, i, 0)),
            pl.BlockSpec((_BR, d), lambda i: (i, 0)),
            pl.BlockSpec((_BR, 2), lambda i: (i, 0)),
            pl.BlockSpec((1, d), lambda i: (0, 0)),
            pl.BlockSpec((d, d), lambda i: (0, 0)),
        ],
        out_specs=pl.BlockSpec((_BR, d), lambda i: (i, 0)),
        out_shape=jax.ShapeDtypeStruct((n, d), jnp.float32),
    )(acc, y1, deg_t, b, w)


def _final_tc(acc, y2, deg_t, b):
    n, d = y2.shape

    def body(acc_ref, y2_ref, deg_ref, b_ref, o_ref):
        dinv = lax.rsqrt(deg_ref[:, :1] + deg_ref[:, 1:2])
        tot = acc_ref[0] + acc_ref[1] + y2_ref[...]
        o_ref[...] = jnp.maximum(tot * dinv + b_ref[...], 0.0)

    return pl.pallas_call(
        body,
        grid=(n // _BR,),
        in_specs=[
            pl.BlockSpec((_NC, _BR, d), lambda i: (0, i, 0)),
            pl.BlockSpec((_BR, d), lambda i: (i, 0)),
            pl.BlockSpec((_BR, 2), lambda i: (i, 0)),
            pl.BlockSpec((1, d), lambda i: (0, 0)),
        ],
        out_specs=pl.BlockSpec((_BR, d), lambda i: (i, 0)),
        out_shape=jax.ShapeDtypeStruct((n, d), jnp.float32),
    )(acc, y2, deg_t, b)


def kernel(edge_index, x, W1, b1, W2, b2):
    n, d = x.shape
    e = edge_index.shape[1]

    nch = 4 * -(-e // (_NW * _K * 4))
    e_pad = _NW * nch * _K
    assert n % _NS == 0 and ((n + _GAR) // 2) % 8 == 0 and n % _BR == 0

    ar = jnp.arange(e_pad - e, dtype=edge_index.dtype)
    pad_blk = jnp.stack([ar % n, n + ar % _GAR], axis=0)
    ei_p = jnp.concatenate([edge_index, pad_blk], axis=1)
    ei3 = (ei_p.reshape(2, e_pad // _K, _K)
           .transpose(1, 0, 2)
           .reshape(e_pad // (4 * _K), 8, _K))

    deg0, deg1 = _degree_sc(ei3, n)
    deg_t = jnp.stack([deg0[:n], deg1[:n]], axis=1)
    b1r = b1.reshape(1, d)
    b2r = b2.reshape(1, d)

    xw1 = _mm_tc(x, W1)
    y1 = _scale_tc(xw1, deg_t)
    acc1 = _segsum_sc(ei3, y1)
    y2 = _mid_layer_tc(acc1, y1, deg_t, b1r, W2)
    acc2 = _segsum_sc(ei3, y2)
    return _final_tc(acc2, y2, deg_t, b2r)
